# baseline (speedup 1.0000x reference)
import functools

import jax
import jax.numpy as jnp
from jax import lax
from jax.experimental import pallas as pl
from jax.experimental.pallas import tpu as pltpu
from jax.experimental.pallas import tpu_sc as plsc

N_NODES = 10000
N_EDGES = 320000
D = 128

NC = 2
NS = 16
NW = NC * NS
NP = 10240
EP = 327680
EPW = EP // NW
STEPS = EPW // 128
BATCH = 128
NBUF = 4
STEPS2 = EPW // BATCH
SPP = STEPS2 // 2
RPW = NP // NW
RPS = NP // NS

_MESH = plsc.VectorSubcoreMesh(core_axis_name="c", subcore_axis_name="s")
_SC_PARAMS = pltpu.CompilerParams(needs_layout_passes=False)


def _rsqrt16(d):
    ibits = plsc.bitcast(d, jnp.int32)
    y = plsc.bitcast(jnp.int32(0x5F3759DF) - (ibits >> 1), jnp.float32)
    half = 0.5 * d
    for _ in range(3):
        y = y * (1.5 - half * y * y)
    return y


@functools.partial(
    pl.kernel,
    mesh=_MESH,
    out_type=(
        jax.ShapeDtypeStruct((NP, D), jnp.float32),
        jax.ShapeDtypeStruct((NC, NS, NP), jnp.float32),
    ),
    scratch_types=[
        pltpu.VMEM((80, 128), jnp.int32),
        pltpu.VMEM((NP,), jnp.float32),
        pltpu.VMEM((NS, RPS), jnp.float32),
        pltpu.VMEM((RPW,), jnp.float32),
        pltpu.VMEM((RPW, D), jnp.float32),
    ],
    compiler_params=_SC_PARAMS,
)
def _deg_dis(dst2_hbm, dism_hbm, cnt_hbm, dbuf, counts, rbuf, disv, obuf):
    cid = lax.axis_index("c")
    sid = lax.axis_index("s")
    wid = sid * NC + cid
    ones = jnp.full((16,), 1.0, jnp.float32)
    zeros16 = jnp.zeros((16,), jnp.float32)

    def zero(i, _):
        counts[pl.ds(i * 16, 16)] = zeros16
        return 0

    lax.fori_loop(0, NP // 16, zero, 0)

    base = sid * (EP // 128 // NS)
    for t in range(EP // 128 // NS // 80):
        pltpu.sync_copy(dst2_hbm.at[pl.ds(base + t * 80, 80)], dbuf)

        def row(r, _):
            for g in range(8):
                idx = dbuf[r, pl.ds(g * 16, 16)]
                plsc.addupdate_scatter(counts, [idx], ones)
            return 0

        lax.fori_loop(0, 80, row, 0)

    pltpu.sync_copy(counts, cnt_hbm.at[cid, sid])
    plsc.subcore_barrier()
    pltpu.sync_copy(cnt_hbm.at[cid, :, pl.ds(sid * RPS, RPS)], rbuf)
    off = cid * RPW
    lo = sid * RPS + off

    def rsq(i, _):
        acc = ones
        for t in range(NS):
            acc = acc + rbuf[t, pl.ds(off + i * 16, 16)]
        disv[pl.ds(i * 16, 16)] = _rsqrt16(acc)
        return 0

    lax.fori_loop(0, RPW // 16, rsq, 0)

    def brow(r, _):
        iv = jnp.zeros((16,), jnp.int32) + r
        s = plsc.load_gather(disv, [iv])
        for g in range(8):
            obuf[r, pl.ds(g * 16, 16)] = s
        return 0

    lax.fori_loop(0, RPW, brow, 0)
    pltpu.sync_copy(obuf, dism_hbm.at[pl.ds(lo, RPW)])


def _mm_body(x_ref, w_ref, d_ref, o_ref):
    o_ref[...] = (
        jnp.dot(x_ref[...], w_ref[...], preferred_element_type=jnp.float32)
        * d_ref[...]
    )


def _matmul_scale(xp, W, dism):
    return pl.pallas_call(
        _mm_body,
        grid=(NP // 1024,),
        in_specs=[
            pl.BlockSpec((1024, D), lambda i: (i, 0)),
            pl.BlockSpec((D, D), lambda i: (0, 0)),
            pl.BlockSpec((1024, D), lambda i: (i, 0)),
        ],
        out_specs=pl.BlockSpec((1024, D), lambda i: (i, 0)),
        out_shape=jax.ShapeDtypeStruct((NP, D), jnp.float32),
    )(xp, W, dism)


@functools.partial(
    pl.kernel,
    mesh=_MESH,
    out_type=jax.ShapeDtypeStruct((NC, NP, D), jnp.float32),
    scratch_types=[
        pltpu.VMEM((SPP, BATCH), jnp.int32),
        pltpu.VMEM((SPP, BATCH), jnp.int32),
        pltpu.VMEM((BATCH, D), jnp.float32),
        pltpu.VMEM((BATCH, D), jnp.float32),
        pltpu.VMEM_SHARED((NP, D), jnp.float32),
        pltpu.SemaphoreType.DMA,
        pltpu.SemaphoreType.DMA,
        pltpu.SemaphoreType.DMA,
        pltpu.SemaphoreType.DMA,
    ],
    compiler_params=_SC_PARAMS,
)
def _scatter(h2_hbm, src_hbm, dst_hbm, z_hbm, out_hbm,
             sidx, didx, gb0, gb1, acc, ga, gbs, sa, sb):
    cid = lax.axis_index("c")
    sid = lax.axis_index("s")
    wid = sid * NC + cid

    pltpu.sync_copy(z_hbm, acc.at[pl.ds(sid * RPS, RPS)])
    plsc.subcore_barrier()

    def wait_gather(buf, sem):
        pltpu.make_async_copy(h2_hbm.at[pl.ds(0, BATCH)], buf, sem).wait()

    def wait_scatter(buf, sem):
        pltpu.make_async_copy(buf, acc.at[didx.at[0]], sem).wait()

    for h in range(STEPS2 // SPP):
        pltpu.sync_copy(src_hbm.at[wid, pl.ds(h * SPP, SPP)], sidx)
        pltpu.sync_copy(dst_hbm.at[wid, pl.ds(h * SPP, SPP)], didx)
        pltpu.async_copy(h2_hbm.at[sidx.at[0]], gb0, ga)
        pltpu.async_copy(h2_hbm.at[sidx.at[1]], gb1, gbs)

        def step(t, _):
            wait_gather(gb0, ga)
            pltpu.async_copy(gb0, acc.at[didx.at[2 * t]], sa, add=True)
            wait_gather(gb1, gbs)
            pltpu.async_copy(gb1, acc.at[didx.at[2 * t + 1]], sb, add=True)
            wait_scatter(gb0, sa)
            wait_scatter(gb1, sb)

            @pl.when(t < SPP // 2 - 1)
            def _():
                pltpu.async_copy(h2_hbm.at[sidx.at[2 * t + 2]], gb0, ga)
                pltpu.async_copy(h2_hbm.at[sidx.at[2 * t + 3]], gb1, gbs)

            return 0

        lax.fori_loop(0, SPP // 2, step, 0)
    plsc.subcore_barrier()
    pltpu.sync_copy(
        acc.at[pl.ds(sid * RPS, RPS)], out_hbm.at[cid, pl.ds(sid * RPS, RPS)]
    )


def _fin_body(s_ref, h_ref, d_ref, b_ref, o_ref):
    s = s_ref[0].astype(jnp.float32) + s_ref[1].astype(jnp.float32)
    t = (s + h_ref[...]) * d_ref[...] + b_ref[0:1, :]
    o_ref[...] = jnp.maximum(t, 0.0)


def _finalize(S, h2, dism, b8):
    return pl.pallas_call(
        _fin_body,
        grid=(NP // 1024,),
        in_specs=[
            pl.BlockSpec((NC, 1024, D), lambda i: (0, i, 0)),
            pl.BlockSpec((1024, D), lambda i: (i, 0)),
            pl.BlockSpec((1024, D), lambda i: (i, 0)),
            pl.BlockSpec((8, D), lambda i: (0, 0)),
        ],
        out_specs=pl.BlockSpec((1024, D), lambda i: (i, 0)),
        out_shape=jax.ShapeDtypeStruct((NP, D), jnp.float32),
    )(S, h2, dism, b8)


def kernel(x, edge_index, W, b):
    ei = edge_index.astype(jnp.int32)
    npad = EP - N_EDGES
    pad_src = (jnp.arange(npad, dtype=jnp.int32) * 13) % NP
    pad_dst = N_NODES + jnp.arange(npad, dtype=jnp.int32) % (NP - N_NODES)
    srcp = jnp.concatenate([ei[0], pad_src]).reshape(NW, STEPS2, BATCH)
    dstp = jnp.concatenate([ei[1], pad_dst])
    dst2 = dstp.reshape(EP // 128, 128)
    dstr = dstp.reshape(NW, STEPS2, BATCH)
    xp = jnp.pad(x.astype(jnp.float32), ((0, NP - N_NODES), (0, 0)))
    zeros = jnp.zeros((RPS, D), jnp.float32)
    b8 = jnp.broadcast_to(b.astype(jnp.float32), (8, D))

    dism, _ = _deg_dis(dst2)
    h2 = _matmul_scale(xp, W.astype(jnp.float32), dism)
    S = _scatter(h2, srcp, dstr, zeros)
    out = _finalize(S, h2, dism, b8)
    return out[:N_NODES]


# speedup vs baseline: 1.2011x; 1.2011x over previous
import functools

import jax
import jax.numpy as jnp
from jax import lax
from jax.experimental import pallas as pl
from jax.experimental.pallas import tpu as pltpu
from jax.experimental.pallas import tpu_sc as plsc

N_NODES = 10000
N_EDGES = 320000
D = 128

NC = 2
NS = 16
NW = NC * NS
NP = 10240
EP = 327680
EPW = EP // NW
STEPS = EPW // 128
BATCH = 128
NBUF = 4
STEPS2 = EPW // BATCH
SPP = STEPS2 // 2
RPW = NP // NW
RPS = NP // NS

_MESH = plsc.VectorSubcoreMesh(core_axis_name="c", subcore_axis_name="s")
_SC_PARAMS = pltpu.CompilerParams(needs_layout_passes=False)


def _rsqrt16(d):
    ibits = plsc.bitcast(d, jnp.int32)
    y = plsc.bitcast(jnp.int32(0x5F3759DF) - (ibits >> 1), jnp.float32)
    half = 0.5 * d
    for _ in range(3):
        y = y * (1.5 - half * y * y)
    return y


@functools.partial(
    pl.kernel,
    mesh=_MESH,
    out_type=(
        jax.ShapeDtypeStruct((NP, D), jnp.float32),
        jax.ShapeDtypeStruct((NC, NS, NP), jnp.float32),
    ),
    scratch_types=[
        pltpu.VMEM((80, 128), jnp.int32),
        pltpu.VMEM((NP,), jnp.float32),
        pltpu.VMEM((NS, RPS), jnp.float32),
        pltpu.VMEM((RPW,), jnp.float32),
        pltpu.VMEM((RPW, D), jnp.float32),
    ],
    compiler_params=_SC_PARAMS,
)
def _deg_dis(dst2_hbm, dism_hbm, cnt_hbm, dbuf, counts, rbuf, disv, obuf):
    cid = lax.axis_index("c")
    sid = lax.axis_index("s")
    wid = sid * NC + cid
    ones = jnp.full((16,), 1.0, jnp.float32)
    zeros16 = jnp.zeros((16,), jnp.float32)

    def zero(i, _):
        counts[pl.ds(i * 16, 16)] = zeros16
        return 0

    lax.fori_loop(0, NP // 16, zero, 0)

    base = sid * (EP // 128 // NS)
    for t in range(EP // 128 // NS // 80):
        pltpu.sync_copy(dst2_hbm.at[pl.ds(base + t * 80, 80)], dbuf)

        def row(r, _):
            for g in range(8):
                idx = dbuf[r, pl.ds(g * 16, 16)]
                plsc.addupdate_scatter(counts, [idx], ones)
            return 0

        lax.fori_loop(0, 80, row, 0)

    pltpu.sync_copy(counts, cnt_hbm.at[cid, sid])
    plsc.subcore_barrier()
    pltpu.sync_copy(cnt_hbm.at[cid, :, pl.ds(sid * RPS, RPS)], rbuf)
    off = cid * RPW
    lo = sid * RPS + off

    def rsq(i, _):
        acc = ones
        for t in range(NS):
            acc = acc + rbuf[t, pl.ds(off + i * 16, 16)]
        disv[pl.ds(i * 16, 16)] = _rsqrt16(acc)
        return 0

    lax.fori_loop(0, RPW // 16, rsq, 0)

    def brow(r, _):
        iv = jnp.zeros((16,), jnp.int32) + r
        s = plsc.load_gather(disv, [iv])
        for g in range(8):
            obuf[r, pl.ds(g * 16, 16)] = s
        return 0

    lax.fori_loop(0, RPW, brow, 0)
    pltpu.sync_copy(obuf, dism_hbm.at[pl.ds(lo, RPW)])


def _mm_body(x_ref, w_ref, d_ref, o_ref):
    o_ref[...] = (
        jnp.dot(x_ref[...], w_ref[...], preferred_element_type=jnp.float32)
        * d_ref[...]
    )


def _matmul_scale(xp, W, dism):
    return pl.pallas_call(
        _mm_body,
        grid=(NP // 1024,),
        in_specs=[
            pl.BlockSpec((1024, D), lambda i: (i, 0)),
            pl.BlockSpec((D, D), lambda i: (0, 0)),
            pl.BlockSpec((1024, D), lambda i: (i, 0)),
        ],
        out_specs=pl.BlockSpec((1024, D), lambda i: (i, 0)),
        out_shape=jax.ShapeDtypeStruct((NP, D), jnp.float32),
    )(xp, W, dism)


@functools.partial(
    pl.kernel,
    mesh=_MESH,
    out_type=jax.ShapeDtypeStruct((NC, NP, D), jnp.float32),
    scratch_types=[
        pltpu.VMEM((SPP, BATCH), jnp.int32),
        pltpu.VMEM((SPP, BATCH), jnp.int32),
        pltpu.VMEM((BATCH, D), jnp.float32),
        pltpu.VMEM((BATCH, D), jnp.float32),
        pltpu.VMEM_SHARED((NP, D), jnp.float32),
        pltpu.SemaphoreType.DMA,
        pltpu.SemaphoreType.DMA,
    ],
    compiler_params=_SC_PARAMS,
)
def _scatter(h2_hbm, src_hbm, dst_hbm, z_hbm, out_hbm,
             sidx, didx, gb0, gb1, acc, semA, semB):
    cid = lax.axis_index("c")
    sid = lax.axis_index("s")
    wid = sid * NC + cid

    pltpu.sync_copy(z_hbm, acc.at[pl.ds(sid * RPS, RPS)])
    plsc.subcore_barrier()

    for h in range(STEPS2 // SPP):
        pltpu.sync_copy(src_hbm.at[wid, pl.ds(h * SPP, SPP)], sidx)
        pltpu.sync_copy(dst_hbm.at[wid, pl.ds(h * SPP, SPP)], didx)
        pltpu.async_copy(h2_hbm.at[sidx.at[0]], gb0, semA)

        def step(t, _):
            pltpu.async_copy(h2_hbm.at[sidx.at[2 * t + 1]], gb1, semB)
            pltpu.make_async_copy(h2_hbm.at[pl.ds(0, BATCH)], gb0, semA).wait()
            pltpu.sync_copy(gb0, acc.at[didx.at[2 * t]], add=True)

            @pl.when(t < SPP // 2 - 1)
            def _():
                pltpu.async_copy(h2_hbm.at[sidx.at[2 * t + 2]], gb0, semA)

            pltpu.make_async_copy(h2_hbm.at[pl.ds(0, BATCH)], gb1, semB).wait()
            pltpu.sync_copy(gb1, acc.at[didx.at[2 * t + 1]], add=True)
            return 0

        lax.fori_loop(0, SPP // 2, step, 0)
    plsc.subcore_barrier()
    pltpu.sync_copy(
        acc.at[pl.ds(sid * RPS, RPS)], out_hbm.at[cid, pl.ds(sid * RPS, RPS)]
    )


def _fin_body(s_ref, h_ref, d_ref, b_ref, o_ref):
    s = s_ref[0].astype(jnp.float32) + s_ref[1].astype(jnp.float32)
    t = (s + h_ref[...]) * d_ref[...] + b_ref[0:1, :]
    o_ref[...] = jnp.maximum(t, 0.0)


def _finalize(S, h2, dism, b8):
    return pl.pallas_call(
        _fin_body,
        grid=(NP // 1024,),
        in_specs=[
            pl.BlockSpec((NC, 1024, D), lambda i: (0, i, 0)),
            pl.BlockSpec((1024, D), lambda i: (i, 0)),
            pl.BlockSpec((1024, D), lambda i: (i, 0)),
            pl.BlockSpec((8, D), lambda i: (0, 0)),
        ],
        out_specs=pl.BlockSpec((1024, D), lambda i: (i, 0)),
        out_shape=jax.ShapeDtypeStruct((NP, D), jnp.float32),
    )(S, h2, dism, b8)


def kernel(x, edge_index, W, b):
    ei = edge_index.astype(jnp.int32)
    npad = EP - N_EDGES
    pad_src = (jnp.arange(npad, dtype=jnp.int32) * 13) % NP
    pad_dst = N_NODES + jnp.arange(npad, dtype=jnp.int32) % (NP - N_NODES)
    srcp = jnp.concatenate([ei[0], pad_src]).reshape(NW, STEPS2, BATCH)
    dstp = jnp.concatenate([ei[1], pad_dst])
    dst2 = dstp.reshape(EP // 128, 128)
    dstr = dstp.reshape(NW, STEPS2, BATCH)
    xp = jnp.pad(x.astype(jnp.float32), ((0, NP - N_NODES), (0, 0)))
    zeros = jnp.zeros((RPS, D), jnp.float32)
    b8 = jnp.broadcast_to(b.astype(jnp.float32), (8, D))

    dism, _ = _deg_dis(dst2)
    h2 = _matmul_scale(xp, W.astype(jnp.float32), dism)
    S = _scatter(h2, srcp, dstr, zeros)
    out = _finalize(S, h2, dism, b8)
    return out[:N_NODES]


# speedup vs baseline: 1.2358x; 1.0288x over previous
import functools

import jax
import jax.numpy as jnp
from jax import lax
from jax.experimental import pallas as pl
from jax.experimental.pallas import tpu as pltpu
from jax.experimental.pallas import tpu_sc as plsc

N_NODES = 10000
N_EDGES = 320000
D = 128

NC = 2
NS = 16
NW = NC * NS
NP = 10240
EP = 327680
EPW = EP // NW
STEPS = EPW // 128
BATCH = 128
NBUF = 4
STEPS2 = EPW // BATCH
SPP = STEPS2 // 2
RPW = NP // NW
RPS = NP // NS

_MESH = plsc.VectorSubcoreMesh(core_axis_name="c", subcore_axis_name="s")
_SC_PARAMS = pltpu.CompilerParams(needs_layout_passes=False)


def _rsqrt16(d):
    ibits = plsc.bitcast(d, jnp.int32)
    y = plsc.bitcast(jnp.int32(0x5F3759DF) - (ibits >> 1), jnp.float32)
    half = 0.5 * d
    for _ in range(3):
        y = y * (1.5 - half * y * y)
    return y


@functools.partial(
    pl.kernel,
    mesh=_MESH,
    out_type=(
        jax.ShapeDtypeStruct((NP, D), jnp.float32),
        jax.ShapeDtypeStruct((NC, NS, NP), jnp.float32),
    ),
    scratch_types=[
        pltpu.VMEM((80, 128), jnp.int32),
        pltpu.VMEM((NP,), jnp.float32),
        pltpu.VMEM((NS, RPS), jnp.float32),
        pltpu.VMEM((RPW,), jnp.float32),
        pltpu.VMEM((RPW, D), jnp.float32),
    ],
    compiler_params=_SC_PARAMS,
)
def _deg_dis(dst2_hbm, dism_hbm, cnt_hbm, dbuf, counts, rbuf, disv, obuf):
    cid = lax.axis_index("c")
    sid = lax.axis_index("s")
    wid = sid * NC + cid
    ones = jnp.full((16,), 1.0, jnp.float32)
    zeros16 = jnp.zeros((16,), jnp.float32)

    def zero(i, _):
        counts[pl.ds(i * 16, 16)] = zeros16
        return 0

    lax.fori_loop(0, NP // 16, zero, 0)

    base = sid * (EP // 128 // NS)
    for t in range(EP // 128 // NS // 80):
        pltpu.sync_copy(dst2_hbm.at[pl.ds(base + t * 80, 80)], dbuf)

        def row(r, _):
            for g in range(8):
                idx = dbuf[r, pl.ds(g * 16, 16)]
                plsc.addupdate_scatter(counts, [idx], ones)
            return 0

        lax.fori_loop(0, 80, row, 0)

    pltpu.sync_copy(counts, cnt_hbm.at[cid, sid])
    plsc.subcore_barrier()
    pltpu.sync_copy(cnt_hbm.at[cid, :, pl.ds(sid * RPS, RPS)], rbuf)
    off = cid * RPW
    lo = sid * RPS + off

    def rsq(i, _):
        acc = ones
        for t in range(NS):
            acc = acc + rbuf[t, pl.ds(off + i * 16, 16)]
        disv[pl.ds(i * 16, 16)] = _rsqrt16(acc)
        return 0

    lax.fori_loop(0, RPW // 16, rsq, 0)

    def brow(r, _):
        iv = jnp.zeros((16,), jnp.int32) + r
        s = plsc.load_gather(disv, [iv])
        for g in range(8):
            obuf[r, pl.ds(g * 16, 16)] = s
        return 0

    lax.fori_loop(0, RPW, brow, 0)
    pltpu.sync_copy(obuf, dism_hbm.at[pl.ds(lo, RPW)])


def _mm_body(x_ref, w_ref, d_ref, o_ref):
    o_ref[...] = (
        jnp.dot(x_ref[...], w_ref[...], preferred_element_type=jnp.float32)
        * d_ref[...]
    )


def _matmul_scale(x, W, dism):
    return pl.pallas_call(
        _mm_body,
        grid=(N_NODES // 1000,),
        in_specs=[
            pl.BlockSpec((1000, D), lambda i: (i, 0)),
            pl.BlockSpec((D, D), lambda i: (0, 0)),
            pl.BlockSpec((1000, D), lambda i: (i, 0)),
        ],
        out_specs=pl.BlockSpec((1000, D), lambda i: (i, 0)),
        out_shape=jax.ShapeDtypeStruct((NP, D), jnp.float32),
    )(x, W, dism)


@functools.partial(
    pl.kernel,
    mesh=_MESH,
    out_type=jax.ShapeDtypeStruct((NC, NP, D), jnp.float32),
    scratch_types=[
        pltpu.VMEM((SPP, BATCH), jnp.int32),
        pltpu.VMEM((SPP, BATCH), jnp.int32),
        pltpu.VMEM((BATCH, D), jnp.float32),
        pltpu.VMEM((BATCH, D), jnp.float32),
        pltpu.VMEM_SHARED((NP, D), jnp.float32),
        pltpu.SemaphoreType.DMA,
        pltpu.SemaphoreType.DMA,
    ],
    compiler_params=_SC_PARAMS,
)
def _scatter(h2_hbm, src_hbm, dst_hbm, z_hbm, out_hbm,
             sidx, didx, gb0, gb1, acc, semA, semB):
    cid = lax.axis_index("c")
    sid = lax.axis_index("s")
    wid = sid * NC + cid

    pltpu.sync_copy(z_hbm, acc.at[pl.ds(sid * RPS, RPS)])
    plsc.subcore_barrier()

    for h in range(STEPS2 // SPP):
        pltpu.sync_copy(src_hbm.at[wid, pl.ds(h * SPP, SPP)], sidx)
        pltpu.sync_copy(dst_hbm.at[wid, pl.ds(h * SPP, SPP)], didx)
        pltpu.async_copy(h2_hbm.at[sidx.at[0]], gb0, semA)

        def step(t, _):
            pltpu.async_copy(h2_hbm.at[sidx.at[2 * t + 1]], gb1, semB)
            pltpu.make_async_copy(h2_hbm.at[pl.ds(0, BATCH)], gb0, semA).wait()
            pltpu.sync_copy(gb0, acc.at[didx.at[2 * t]], add=True)

            @pl.when(t < SPP // 2 - 1)
            def _():
                pltpu.async_copy(h2_hbm.at[sidx.at[2 * t + 2]], gb0, semA)

            pltpu.make_async_copy(h2_hbm.at[pl.ds(0, BATCH)], gb1, semB).wait()
            pltpu.sync_copy(gb1, acc.at[didx.at[2 * t + 1]], add=True)
            return 0

        lax.fori_loop(0, SPP // 2, step, 0)
    plsc.subcore_barrier()
    pltpu.sync_copy(
        acc.at[pl.ds(sid * RPS, RPS)], out_hbm.at[cid, pl.ds(sid * RPS, RPS)]
    )


def _fin_body(s_ref, h_ref, d_ref, b_ref, o_ref):
    s = s_ref[0].astype(jnp.float32) + s_ref[1].astype(jnp.float32)
    t = (s + h_ref[...]) * d_ref[...] + b_ref[0:1, :]
    o_ref[...] = jnp.maximum(t, 0.0)


def _finalize(S, h2, dism, b8):
    return pl.pallas_call(
        _fin_body,
        grid=(N_NODES // 1000,),
        in_specs=[
            pl.BlockSpec((NC, 1000, D), lambda i: (0, i, 0)),
            pl.BlockSpec((1000, D), lambda i: (i, 0)),
            pl.BlockSpec((1000, D), lambda i: (i, 0)),
            pl.BlockSpec((8, D), lambda i: (0, 0)),
        ],
        out_specs=pl.BlockSpec((1000, D), lambda i: (i, 0)),
        out_shape=jax.ShapeDtypeStruct((N_NODES, D), jnp.float32),
    )(S, h2, dism, b8)


def kernel(x, edge_index, W, b):
    ei = edge_index.astype(jnp.int32)
    npad = EP - N_EDGES
    pad = N_NODES + jnp.arange(npad, dtype=jnp.int32) % (NP - N_NODES)
    srcp = jnp.concatenate([ei[0], pad]).reshape(NW, STEPS2, BATCH)
    dstp = jnp.concatenate([ei[1], pad])
    dst2 = dstp.reshape(EP // 128, 128)
    dstr = dstp.reshape(NW, STEPS2, BATCH)
    zeros = jnp.zeros((RPS, D), jnp.float32)
    b8 = jnp.broadcast_to(b.astype(jnp.float32), (8, D))

    dism, _ = _deg_dis(dst2)
    h2 = _matmul_scale(x.astype(jnp.float32), W.astype(jnp.float32), dism)
    S = _scatter(h2, srcp, dstr, zeros)
    return _finalize(S, h2, dism, b8)
